# baseline (speedup 1.0000x reference)
import functools

import jax
import jax.numpy as jnp
from jax import lax
from jax.experimental import pallas as pl
from jax.experimental.pallas import tpu as pltpu
from jax.experimental.pallas import tpu_sc as plsc

TB = 512


def _sc_lengths(mask_flat, B, T):
    mesh = plsc.VectorSubcoreMesh(core_axis_name="c", subcore_axis_name="s")

    @functools.partial(
        pl.kernel,
        mesh=mesh,
        compiler_params=pltpu.CompilerParams(needs_layout_passes=False),
        out_type=[jax.ShapeDtypeStruct((B * 8,), jnp.int32)],
        scratch_types=[
            pltpu.VMEM((T,), jnp.int32),
            pltpu.VMEM((16,), jnp.int32),
        ],
    )
    def body(m_hbm, len_hbm, mbuf, lenbuf):
        c = lax.axis_index("c")
        s = lax.axis_index("s")
        wid = s * 2 + c
        b = wid // 2
        h = wid % 2

        @pl.when(h == 0)
        def _():
            pltpu.sync_copy(m_hbm.at[pl.ds(b * T, T)], mbuf)

            def sbody(i, acc):
                j = i * 64
                return (acc + mbuf[pl.ds(j, 16)] + mbuf[pl.ds(j + 16, 16)]
                        + mbuf[pl.ds(j + 32, 16)] + mbuf[pl.ds(j + 48, 16)])

            acc = lax.fori_loop(0, T // 64, sbody, jnp.zeros((16,), jnp.int32))
            len0 = T - jnp.sum(acc)
            lenbuf[...] = jnp.where(
                lax.broadcasted_iota(jnp.int32, (16,), 0) == 0, len0 + 2, 0)
            pltpu.sync_copy(lenbuf.at[pl.ds(0, 8)], len_hbm.at[pl.ds(b * 8, 8)])

    return body(mask_flat)


def _tc_assemble(x, bos_emb, eos_emb, lengths, B, T, C):
    To = T + 2
    NI = (To + TB - 1) // TB
    NXB = T // TB

    def tc_body(len_ref, prev_ref, cur_ref, bos_ref, eos_ref, out_ref):
        b = pl.program_id(0)
        i = pl.program_id(1)
        p0 = i * TB
        len0 = len_ref[b] - 2
        E = len0 + 1

        def rotated():
            cur = cur_ref[0]
            prev_last = prev_ref[0, 7:8, :]
            return jnp.concatenate([prev_last, cur[:TB - 1, :]], axis=0)

        all_copy = (p0 + TB <= E) & (p0 >= 1)
        all_zero = p0 >= E + 2

        @pl.when(all_copy)
        def _():
            out_ref[0] = rotated()

        @pl.when(all_zero)
        def _():
            out_ref[0] = jnp.zeros((TB, C), jnp.float32)

        @pl.when(jnp.logical_not(all_copy | all_zero))
        def _():
            rows = p0 + lax.broadcasted_iota(jnp.int32, (TB, C), 0)
            shifted = rotated()
            keep = (rows >= 1) & (rows <= len0)
            val = jnp.where(keep, shifted, 0.0)
            val = jnp.where(rows == E,
                            jnp.broadcast_to(eos_ref[:][None, :], (TB, C)), val)
            val = jnp.where(rows == 0,
                            jnp.broadcast_to(bos_ref[:][None, :], (TB, C)), val)
            out_ref[0] = val

    grid_spec = pltpu.PrefetchScalarGridSpec(
        num_scalar_prefetch=1,
        grid=(B, NI),
        in_specs=[
            pl.BlockSpec((1, 8, C),
                         lambda b, i, L: (b, jnp.maximum(i * (TB // 8) - 1, 0), 0)),
            pl.BlockSpec((1, TB, C),
                         lambda b, i, L: (b, jnp.minimum(i, NXB - 1), 0)),
            pl.BlockSpec((C,), lambda b, i, L: (0,)),
            pl.BlockSpec((C,), lambda b, i, L: (0,)),
        ],
        out_specs=pl.BlockSpec((1, TB, C), lambda b, i, L: (b, i, 0)),
    )
    return pl.pallas_call(
        tc_body,
        grid_spec=grid_spec,
        out_shape=jax.ShapeDtypeStruct((B, To, C), jnp.float32),
    )(lengths, x, x, bos_emb, eos_emb)


def kernel(x, bos_emb, eos_emb, padding_mask):
    B, T, C = x.shape
    mask_flat = padding_mask.astype(jnp.int32).reshape(B * T)
    [len_pad] = _sc_lengths(mask_flat, B, T)
    lengths = len_pad.reshape(B, 8)[:, 0]
    xe = _tc_assemble(x, bos_emb, eos_emb, lengths, B, T, C)
    new_padding_mask = jnp.arange(T + 2)[None, :] >= lengths[:, None]
    return (xe, new_padding_mask, lengths)


# speedup vs baseline: 1.0128x; 1.0128x over previous
import functools

import jax
import jax.numpy as jnp
from jax import lax
from jax.experimental import pallas as pl
from jax.experimental.pallas import tpu as pltpu
from jax.experimental.pallas import tpu_sc as plsc

TB = 512


def _sc_lengths(mask_flat, B, T):
    mesh = plsc.VectorSubcoreMesh(core_axis_name="c", subcore_axis_name="s")

    @functools.partial(
        pl.kernel,
        mesh=mesh,
        compiler_params=pltpu.CompilerParams(needs_layout_passes=False),
        out_type=[jax.ShapeDtypeStruct((B * 8,), jnp.int32)],
        scratch_types=[
            pltpu.VMEM((T,), jnp.int32),
            pltpu.VMEM((16,), jnp.int32),
        ],
    )
    def body(m_hbm, len_hbm, mbuf, lenbuf):
        c = lax.axis_index("c")
        s = lax.axis_index("s")
        wid = s * 2 + c
        b = wid // 2
        h = wid % 2

        @pl.when(h == 0)
        def _():
            pltpu.sync_copy(m_hbm.at[pl.ds(b * T, T)], mbuf)

            def sbody(i, acc):
                j = i * 64
                return (acc + mbuf[pl.ds(j, 16)] + mbuf[pl.ds(j + 16, 16)]
                        + mbuf[pl.ds(j + 32, 16)] + mbuf[pl.ds(j + 48, 16)])

            acc = lax.fori_loop(0, T // 64, sbody, jnp.zeros((16,), jnp.int32))
            len0 = T - jnp.sum(acc)
            lenbuf[...] = jnp.where(
                lax.broadcasted_iota(jnp.int32, (16,), 0) == 0, len0 + 2, 0)
            pltpu.sync_copy(lenbuf.at[pl.ds(0, 8)], len_hbm.at[pl.ds(b * 8, 8)])

    return body(mask_flat)


def _tc_assemble(x, bos_emb, eos_emb, lengths, B, T, C):
    To = T + 2
    NI = (To + TB - 1) // TB
    NXB = T // TB

    def tc_body(len_ref, prev_ref, cur_ref, bos_ref, eos_ref, out_ref):
        b = pl.program_id(0)
        i = pl.program_id(1)
        p0 = i * TB
        len0 = len_ref[b] - 2
        E = len0 + 1

        def rotated():
            cur = cur_ref[0]
            prev_last = prev_ref[0, 7:8, :]
            return jnp.concatenate([prev_last, cur[:TB - 1, :]], axis=0)

        all_copy = (p0 + TB <= E) & (p0 >= 1)
        all_zero = p0 >= E + 2

        @pl.when(all_copy)
        def _():
            out_ref[0] = rotated()

        @pl.when(all_zero)
        def _():
            out_ref[0] = jnp.zeros((TB, C), jnp.float32)

        @pl.when(jnp.logical_not(all_copy | all_zero))
        def _():
            rows = p0 + lax.broadcasted_iota(jnp.int32, (TB, C), 0)
            shifted = rotated()
            keep = (rows >= 1) & (rows <= len0)
            val = jnp.where(keep, shifted, 0.0)
            val = jnp.where(rows == E,
                            jnp.broadcast_to(eos_ref[:][None, :], (TB, C)), val)
            val = jnp.where(rows == 0,
                            jnp.broadcast_to(bos_ref[:][None, :], (TB, C)), val)
            out_ref[0] = val

    grid_spec = pltpu.PrefetchScalarGridSpec(
        num_scalar_prefetch=1,
        grid=(B, NI),
        in_specs=[
            pl.BlockSpec((1, 8, C),
                         lambda b, i, L: (b, jnp.maximum(i * (TB // 8) - 1, 0), 0)),
            pl.BlockSpec((1, TB, C),
                         lambda b, i, L: (b, jnp.where(
                             i * TB >= L[b] + 1,
                             jnp.minimum((L[b] - 1) // TB, NXB - 1),
                             jnp.minimum(i, NXB - 1)), 0)),
            pl.BlockSpec((C,), lambda b, i, L: (0,)),
            pl.BlockSpec((C,), lambda b, i, L: (0,)),
        ],
        out_specs=pl.BlockSpec((1, TB, C), lambda b, i, L: (b, i, 0)),
    )
    return pl.pallas_call(
        tc_body,
        grid_spec=grid_spec,
        out_shape=jax.ShapeDtypeStruct((B, To, C), jnp.float32),
    )(lengths, x, x, bos_emb, eos_emb)


def kernel(x, bos_emb, eos_emb, padding_mask):
    B, T, C = x.shape
    mask_flat = padding_mask.astype(jnp.int32).reshape(B * T)
    [len_pad] = _sc_lengths(mask_flat, B, T)
    lengths = len_pad.reshape(B, 8)[:, 0]
    xe = _tc_assemble(x, bos_emb, eos_emb, lengths, B, T, C)
    new_padding_mask = jnp.arange(T + 2)[None, :] >= lengths[:, None]
    return (xe, new_padding_mask, lengths)


# speedup vs baseline: 1.0376x; 1.0245x over previous
import functools

import jax
import jax.numpy as jnp
from jax import lax
from jax.experimental import pallas as pl
from jax.experimental.pallas import tpu as pltpu
from jax.experimental.pallas import tpu_sc as plsc

TB = 1024


def _sc_lengths(mask_flat, B, T):
    mesh = plsc.VectorSubcoreMesh(core_axis_name="c", subcore_axis_name="s")

    @functools.partial(
        pl.kernel,
        mesh=mesh,
        compiler_params=pltpu.CompilerParams(needs_layout_passes=False),
        out_type=[jax.ShapeDtypeStruct((B * 8,), jnp.int32)],
        scratch_types=[
            pltpu.VMEM((T,), jnp.int32),
            pltpu.VMEM((16,), jnp.int32),
        ],
    )
    def body(m_hbm, len_hbm, mbuf, lenbuf):
        c = lax.axis_index("c")
        s = lax.axis_index("s")
        wid = s * 2 + c
        b = wid // 2
        h = wid % 2

        @pl.when(h == 0)
        def _():
            pltpu.sync_copy(m_hbm.at[pl.ds(b * T, T)], mbuf)

            def sbody(i, acc):
                j = i * 64
                return (acc + mbuf[pl.ds(j, 16)] + mbuf[pl.ds(j + 16, 16)]
                        + mbuf[pl.ds(j + 32, 16)] + mbuf[pl.ds(j + 48, 16)])

            acc = lax.fori_loop(0, T // 64, sbody, jnp.zeros((16,), jnp.int32))
            len0 = T - jnp.sum(acc)
            lenbuf[...] = jnp.where(
                lax.broadcasted_iota(jnp.int32, (16,), 0) == 0, len0 + 2, 0)
            pltpu.sync_copy(lenbuf.at[pl.ds(0, 8)], len_hbm.at[pl.ds(b * 8, 8)])

    return body(mask_flat)


def _tc_assemble(x, bos_emb, eos_emb, lengths, B, T, C):
    To = T + 2
    NI = (To + TB - 1) // TB
    NXB = T // TB

    def tc_body(len_ref, cur_ref, bos_ref, eos_ref, out_ref, carry_ref):
        b = pl.program_id(0)
        i = pl.program_id(1)
        p0 = i * TB
        len0 = len_ref[b] - 2
        E = len0 + 1

        def rotated():
            cur = cur_ref[0]
            prev_last = carry_ref[...]
            return jnp.concatenate([prev_last, cur[:TB - 1, :]], axis=0)

        all_copy = (p0 + TB <= E) & (p0 >= 1)
        all_zero = p0 >= E + 2

        @pl.when(all_copy)
        def _():
            out_ref[0] = rotated()

        @pl.when(all_zero)
        def _():
            out_ref[0] = jnp.zeros((TB, C), jnp.float32)

        @pl.when(jnp.logical_not(all_copy | all_zero))
        def _():
            rows = p0 + lax.broadcasted_iota(jnp.int32, (TB, C), 0)
            shifted = rotated()
            keep = (rows >= 1) & (rows <= len0)
            val = jnp.where(keep, shifted, 0.0)
            val = jnp.where(rows == E,
                            jnp.broadcast_to(eos_ref[:][None, :], (TB, C)), val)
            val = jnp.where(rows == 0,
                            jnp.broadcast_to(bos_ref[:][None, :], (TB, C)), val)
            out_ref[0] = val

        carry_ref[...] = cur_ref[0, TB - 1:TB, :]

    grid_spec = pltpu.PrefetchScalarGridSpec(
        num_scalar_prefetch=1,
        grid=(B, NI),
        in_specs=[
            pl.BlockSpec((1, TB, C),
                         lambda b, i, L: (b, jnp.where(
                             i * TB >= L[b] + 1,
                             jnp.minimum((L[b] - 1) // TB, NXB - 1),
                             jnp.minimum(i, NXB - 1)), 0)),
            pl.BlockSpec((C,), lambda b, i, L: (0,)),
            pl.BlockSpec((C,), lambda b, i, L: (0,)),
        ],
        out_specs=pl.BlockSpec((1, TB, C), lambda b, i, L: (b, i, 0)),
        scratch_shapes=[pltpu.VMEM((1, C), jnp.float32)],
    )
    return pl.pallas_call(
        tc_body,
        grid_spec=grid_spec,
        out_shape=jax.ShapeDtypeStruct((B, To, C), jnp.float32),
    )(lengths, x, bos_emb, eos_emb)


def kernel(x, bos_emb, eos_emb, padding_mask):
    B, T, C = x.shape
    mask_flat = padding_mask.astype(jnp.int32).reshape(B * T)
    [len_pad] = _sc_lengths(mask_flat, B, T)
    lengths = len_pad.reshape(B, 8)[:, 0]
    xe = _tc_assemble(x, bos_emb, eos_emb, lengths, B, T, C)
    new_padding_mask = jnp.arange(T + 2)[None, :] >= lengths[:, None]
    return (xe, new_padding_mask, lengths)
